# baseline (speedup 1.0000x reference)
import functools

import jax
import jax.numpy as jnp
from jax import lax
from jax.experimental import pallas as pl
from jax.experimental.pallas import tpu as pltpu
from jax.experimental.pallas import tpu_sc as plsc

RES = 256
P = RES * RES
PB = 256


def _sc_gather(table, idx):
    B = idx.shape[0]
    D = table.shape[1]
    n_workers = 32
    bpw = B // n_workers
    n_chunks = 2
    cw = bpw // n_chunks
    mesh = plsc.VectorSubcoreMesh(core_axis_name="c", subcore_axis_name="s")

    @functools.partial(
        pl.kernel,
        mesh=mesh,
        out_type=jax.ShapeDtypeStruct((B, D), jnp.float32),
        scratch_types=[
            pltpu.VMEM((n_chunks, cw), jnp.int32),
            pltpu.VMEM((cw, D), jnp.float32),
            pltpu.SemaphoreType.DMA,
        ],
    )
    def k(table_hbm, idx_hbm, out_hbm, idx_v, rows_v, sem):
        wid = lax.axis_index("s") * 2 + lax.axis_index("c")
        base = wid * bpw
        for j in range(n_chunks):
            pltpu.sync_copy(idx_hbm.at[pl.ds(base + j * cw, cw)], idx_v.at[j])
            pltpu.async_copy(table_hbm.at[idx_v.at[j]], rows_v, sem).wait()
            pltpu.sync_copy(rows_v, out_hbm.at[pl.ds(base + j * cw, cw)])

    return k(table, idx)


def _raster_body(vdata_ref, a0_ref, a1_ref, a2_ref, out_ref):
    F = a0_ref.shape[0]
    v0x = vdata_ref[0, 0:1, :]
    v0y = vdata_ref[0, 1:2, :]
    v1x = vdata_ref[1, 0:1, :]
    v1y = vdata_ref[1, 1:2, :]
    v2x = vdata_ref[2, 0:1, :]
    v2y = vdata_ref[2, 1:2, :]

    e0 = v1y - v2y
    e1 = v2x - v1x
    e2 = v2y - v0y
    e3 = v0x - v2x
    d = e0 * (v0x - v2x) + e1 * (v0y - v2y)
    valid = jnp.abs(d) > 1e-8
    absd = jnp.abs(d)
    s = jnp.where(d >= 0.0, 1.0, -1.0)
    cu = jnp.where(valid, -s * (e0 * v2x + e1 * v2y), -1.0)
    cv = jnp.where(valid, -s * (e2 * v2x + e3 * v2y), 0.0)
    ku0 = jnp.where(valid, s * e0, 0.0)
    ku1 = jnp.where(valid, s * e1, 0.0)
    kv0 = jnp.where(valid, s * e2, 0.0)
    kv1 = jnp.where(valid, s * e3, 0.0)
    coef = jnp.concatenate(
        [
            jnp.concatenate([ku0, kv0], axis=1),
            jnp.concatenate([ku1, kv1], axis=1),
            jnp.concatenate([cu, cv], axis=1),
        ],
        axis=0,
    )

    pid = pl.program_id(0)
    p = pid * PB + lax.broadcasted_iota(jnp.int32, (PB, 1), 0)
    pxs = ((p & (RES - 1)).astype(jnp.float32) + 0.5) / float(RES)
    pys = ((p >> 8).astype(jnp.float32) + 0.5) / float(RES)
    ones = jnp.ones((PB, 1), jnp.float32)
    pix = jnp.concatenate([pxs, pys, ones], axis=1)

    suv = lax.dot(pix, coef, precision=lax.Precision.HIGHEST)
    su = suv[:, :F]
    sv = suv[:, F:]
    sw = absd - su - sv
    min3 = jnp.minimum(su, jnp.minimum(sv, sw))

    cols = lax.broadcasted_iota(jnp.int32, (PB, F), 1)
    fidx = jnp.min(
        jnp.where(min3 >= 0.0, cols, jnp.int32(1 << 30)), axis=1, keepdims=True
    )
    oh = cols == fidx
    rd = jnp.where(valid, 1.0 / absd, 1.0)
    ohf = jnp.where(oh, rd, 0.0)
    m0 = ohf * su
    m1 = ohf * sv
    m2 = ohf * sw

    acc = lax.dot(m0, a0_ref[...], precision=lax.Precision.HIGHEST)
    acc += lax.dot(m1, a1_ref[...], precision=lax.Precision.HIGHEST)
    acc += lax.dot(m2, a2_ref[...], precision=lax.Precision.HIGHEST)
    out_ref[...] = acc


def _bake(vdata, a0, a1, a2, interpret=False):
    F = a0.shape[0]
    return pl.pallas_call(
        _raster_body,
        grid=(P // PB,),
        in_specs=[
            pl.BlockSpec((3, 8, F), lambda i: (0, 0, 0)),
            pl.BlockSpec((F, 3), lambda i: (0, 0)),
            pl.BlockSpec((F, 3), lambda i: (0, 0)),
            pl.BlockSpec((F, 3), lambda i: (0, 0)),
        ],
        out_specs=pl.BlockSpec((PB, 3), lambda i: (i, 0)),
        out_shape=jax.ShapeDtypeStruct((P, 3), jnp.float32),
        compiler_params=pltpu.CompilerParams(
            dimension_semantics=("parallel",),
        ),
        interpret=interpret,
    )(vdata, a0, a1, a2)


def kernel(attr, uv, face_indices, bake_resolution, device):
    V = uv.shape[0]
    F = face_indices.shape[0]
    table = jnp.concatenate(
        [uv, attr, jnp.zeros((V, 123), jnp.float32)], axis=1
    )
    idx = face_indices.astype(jnp.int32).T.reshape(-1)
    g = _sc_gather(table, idx).reshape(3, F, 128)
    vdata = jnp.transpose(g[:, :, :8], (0, 2, 1))
    a0 = g[0, :, 2:5]
    a1 = g[1, :, 2:5]
    a2 = g[2, :, 2:5]
    out = _bake(vdata, a0, a1, a2)
    return out.reshape(RES, RES, 3)


# speedup vs baseline: 3.3250x; 3.3250x over previous
import functools

import jax
import jax.numpy as jnp
from jax import lax
from jax.experimental import pallas as pl
from jax.experimental.pallas import tpu as pltpu
from jax.experimental.pallas import tpu_sc as plsc

RES = 256
P = RES * RES
PB = 256
FC = 512
BIG = 1 << 30


def _sc_gather(table, idx):
    B = idx.shape[0]
    D = table.shape[1]
    n_workers = 32
    bpw = B // n_workers
    n_chunks = 2
    cw = bpw // n_chunks
    mesh = plsc.VectorSubcoreMesh(core_axis_name="c", subcore_axis_name="s")

    @functools.partial(
        pl.kernel,
        mesh=mesh,
        out_type=jax.ShapeDtypeStruct((B, D), jnp.float32),
        scratch_types=[
            pltpu.VMEM((n_chunks, cw), jnp.int32),
            pltpu.VMEM((cw, D), jnp.float32),
            pltpu.SemaphoreType.DMA,
        ],
    )
    def k(table_hbm, idx_hbm, out_hbm, idx_v, rows_v, sem):
        wid = lax.axis_index("s") * 2 + lax.axis_index("c")
        base = wid * bpw
        for j in range(n_chunks):
            pltpu.sync_copy(idx_hbm.at[pl.ds(base + j * cw, cw)], idx_v.at[j])
            pltpu.async_copy(table_hbm.at[idx_v.at[j]], rows_v, sem).wait()
            pltpu.sync_copy(rows_v, out_hbm.at[pl.ds(base + j * cw, cw)])

    return k(table, idx)


def _raster_body(
    vdata_ref, a0_ref, a1_ref, a2_ref, out_ref, fidx_s, acc_s, done_s
):
    c = pl.program_id(1)
    nc = pl.num_programs(1)

    @pl.when(c == 0)
    def _init():
        fidx_s[...] = jnp.full((PB, 1), BIG, jnp.int32)
        acc_s[...] = jnp.zeros((PB, 3), jnp.float32)
        done_s[0] = 0

    @pl.when((c == 0) | (done_s[0] == 0))
    def _chunk():
        v0x = vdata_ref[0, 0:1, :]
        v0y = vdata_ref[0, 1:2, :]
        v1x = vdata_ref[1, 0:1, :]
        v1y = vdata_ref[1, 1:2, :]
        v2x = vdata_ref[2, 0:1, :]
        v2y = vdata_ref[2, 1:2, :]

        e0 = v1y - v2y
        e1 = v2x - v1x
        e2 = v2y - v0y
        e3 = v0x - v2x
        d = e0 * (v0x - v2x) + e1 * (v0y - v2y)
        absd = jnp.abs(d)
        valid = absd > 1e-8
        s = jnp.where(d >= 0.0, 1.0, -1.0)
        ku0 = s * e0
        ku1 = s * e1
        kv0 = s * e2
        kv1 = s * e3
        absd_x = jnp.where(valid, absd, -1.0)
        rd = jnp.where(valid, 1.0 / absd, 1.0)

        pid = pl.program_id(0)
        p = pid * PB + lax.broadcasted_iota(jnp.int32, (PB, 1), 0)
        pxs = ((p & (RES - 1)).astype(jnp.float32) + 0.5) / float(RES)
        pys = ((p >> 8).astype(jnp.float32) + 0.5) / float(RES)

        t0 = pxs - v2x
        t1 = pys - v2y
        su = ku0 * t0 + ku1 * t1
        sv = kv0 * t0 + kv1 * t1
        sw = absd_x - su - sv
        min3 = jnp.minimum(su, jnp.minimum(sv, sw))

        cols = lax.broadcasted_iota(jnp.int32, (PB, FC), 1)
        localmin = jnp.min(
            jnp.where(min3 >= 0.0, cols, BIG), axis=1, keepdims=True
        )
        fidxc = localmin + c * FC
        fold = fidx_s[...]
        upd = fidxc < fold
        ohf = jnp.where((cols == localmin) & upd, rd, 0.0)
        m0 = ohf * su
        m1 = ohf * sv
        m2 = ohf * sw

        acc = lax.dot(m0, a0_ref[...])
        acc += lax.dot(m1, a1_ref[...])
        acc += lax.dot(m2, a2_ref[...])
        acc_s[...] += acc
        fnew = jnp.where(upd, fidxc, fold)
        fidx_s[...] = fnew
        done_s[0] = (jnp.max(fnew) < BIG).astype(jnp.int32)

    @pl.when(c == nc - 1)
    def _fin():
        out_ref[...] = acc_s[...]


def _bake(vdata, a0, a1, a2, interpret=False):
    F = a0.shape[0]
    return pl.pallas_call(
        _raster_body,
        grid=(P // PB, F // FC),
        in_specs=[
            pl.BlockSpec((3, 8, FC), lambda b, c: (0, 0, c)),
            pl.BlockSpec((FC, 3), lambda b, c: (c, 0)),
            pl.BlockSpec((FC, 3), lambda b, c: (c, 0)),
            pl.BlockSpec((FC, 3), lambda b, c: (c, 0)),
        ],
        out_specs=pl.BlockSpec((PB, 3), lambda b, c: (b, 0)),
        out_shape=jax.ShapeDtypeStruct((P, 3), jnp.float32),
        scratch_shapes=[
            pltpu.VMEM((PB, 1), jnp.int32),
            pltpu.VMEM((PB, 3), jnp.float32),
            pltpu.SMEM((1,), jnp.int32),
        ],
        compiler_params=pltpu.CompilerParams(
            dimension_semantics=("arbitrary", "arbitrary"),
        ),
        interpret=interpret,
    )(vdata, a0, a1, a2)


def kernel(attr, uv, face_indices, bake_resolution, device):
    V = uv.shape[0]
    F = face_indices.shape[0]
    table = jnp.concatenate(
        [uv, attr, jnp.zeros((V, 123), jnp.float32)], axis=1
    )
    idx = face_indices.astype(jnp.int32).T.reshape(-1)
    g = _sc_gather(table, idx).reshape(3, F, 128)
    vdata = jnp.transpose(g[:, :, :8], (0, 2, 1))
    a0 = g[0, :, 2:5]
    a1 = g[1, :, 2:5]
    a2 = g[2, :, 2:5]
    out = _bake(vdata, a0, a1, a2)
    return out.reshape(RES, RES, 3)


# speedup vs baseline: 4.6777x; 1.4068x over previous
import functools

import jax
import jax.numpy as jnp
from jax import lax
from jax.experimental import pallas as pl
from jax.experimental.pallas import tpu as pltpu
from jax.experimental.pallas import tpu_sc as plsc

RES = 256
P = RES * RES
PB = 1024
FC = 256
BIG = 1 << 30


def _sc_gather(table, idx):
    B = idx.shape[0]
    D = table.shape[1]
    n_workers = 32
    bpw = B // n_workers
    n_chunks = 2
    cw = bpw // n_chunks
    mesh = plsc.VectorSubcoreMesh(core_axis_name="c", subcore_axis_name="s")

    @functools.partial(
        pl.kernel,
        mesh=mesh,
        out_type=jax.ShapeDtypeStruct((B, D), jnp.float32),
        scratch_types=[
            pltpu.VMEM((n_chunks, cw), jnp.int32),
            pltpu.VMEM((cw, D), jnp.float32),
            pltpu.SemaphoreType.DMA,
        ],
    )
    def k(table_hbm, idx_hbm, out_hbm, idx_v, rows_v, sem):
        wid = lax.axis_index("s") * 2 + lax.axis_index("c")
        base = wid * bpw
        for j in range(n_chunks):
            pltpu.sync_copy(idx_hbm.at[pl.ds(base + j * cw, cw)], idx_v.at[j])
            pltpu.async_copy(table_hbm.at[idx_v.at[j]], rows_v, sem).wait()
            pltpu.sync_copy(rows_v, out_hbm.at[pl.ds(base + j * cw, cw)])

    return k(table, idx)


def _raster_body(
    vdata_ref, a0_ref, a1_ref, a2_ref, out_ref, fidx_s, acc_s, done_s
):
    nc = a0_ref.shape[0]
    fidx_s[...] = jnp.full((PB, 1), BIG, jnp.int32)
    acc_s[...] = jnp.zeros((PB, 3), jnp.float32)
    done_s[0] = 0

    pid = pl.program_id(0)
    p = pid * PB + lax.broadcasted_iota(jnp.int32, (PB, 1), 0)
    pxs = ((p & (RES - 1)).astype(jnp.float32) + 0.5) / float(RES)
    pys = ((p >> 8).astype(jnp.float32) + 0.5) / float(RES)

    def chunk(cidx, carry):
        @pl.when(done_s[0] == 0)
        def _():
            v0x = vdata_ref[0, cidx, 0:1, :]
            v0y = vdata_ref[0, cidx, 1:2, :]
            v1x = vdata_ref[1, cidx, 0:1, :]
            v1y = vdata_ref[1, cidx, 1:2, :]
            v2x = vdata_ref[2, cidx, 0:1, :]
            v2y = vdata_ref[2, cidx, 1:2, :]

            e0 = v1y - v2y
            e1 = v2x - v1x
            e2 = v2y - v0y
            e3 = v0x - v2x
            d = e0 * (v0x - v2x) + e1 * (v0y - v2y)
            absd = jnp.abs(d)
            valid = absd > 1e-8
            s = jnp.where(d >= 0.0, 1.0, -1.0)
            ku0 = s * e0
            ku1 = s * e1
            kv0 = s * e2
            kv1 = s * e3
            absd_x = jnp.where(valid, absd, -1.0)
            rd = jnp.where(valid, 1.0 / absd, 1.0)

            t0 = pxs - v2x
            t1 = pys - v2y
            su = ku0 * t0 + ku1 * t1
            sv = kv0 * t0 + kv1 * t1
            sw = absd_x - su - sv
            min3 = jnp.minimum(su, jnp.minimum(sv, sw))

            cols = lax.broadcasted_iota(jnp.int32, (PB, FC), 1)
            localmin = jnp.min(
                jnp.where(min3 >= 0.0, cols, BIG), axis=1, keepdims=True
            )
            fidxc = localmin + cidx * FC
            fold = fidx_s[...]
            upd = fidxc < fold
            ohf = jnp.where((cols == localmin) & upd, rd, 0.0)
            m0 = ohf * su
            m1 = ohf * sv
            m2 = ohf * sw

            acc = lax.dot(m0, a0_ref[cidx])
            acc += lax.dot(m1, a1_ref[cidx])
            acc += lax.dot(m2, a2_ref[cidx])
            acc_s[...] += acc
            fnew = jnp.where(upd, fidxc, fold)
            fidx_s[...] = fnew
            done_s[0] = (jnp.max(fnew) < BIG).astype(jnp.int32)

        return carry

    lax.fori_loop(0, nc, chunk, 0)
    out_ref[...] = acc_s[...]


def _bake(vdata, a0, a1, a2, interpret=False):
    nc = a0.shape[0]
    return pl.pallas_call(
        _raster_body,
        grid=(P // PB,),
        in_specs=[
            pl.BlockSpec((3, nc, 8, FC), lambda b: (0, 0, 0, 0)),
            pl.BlockSpec((nc, FC, 3), lambda b: (0, 0, 0)),
            pl.BlockSpec((nc, FC, 3), lambda b: (0, 0, 0)),
            pl.BlockSpec((nc, FC, 3), lambda b: (0, 0, 0)),
        ],
        out_specs=pl.BlockSpec((PB, 3), lambda b: (b, 0)),
        out_shape=jax.ShapeDtypeStruct((P, 3), jnp.float32),
        scratch_shapes=[
            pltpu.VMEM((PB, 1), jnp.int32),
            pltpu.VMEM((PB, 3), jnp.float32),
            pltpu.SMEM((1,), jnp.int32),
        ],
        compiler_params=pltpu.CompilerParams(
            dimension_semantics=("arbitrary",),
        ),
        interpret=interpret,
    )(vdata, a0, a1, a2)


def kernel(attr, uv, face_indices, bake_resolution, device):
    V = uv.shape[0]
    F = face_indices.shape[0]
    nc = F // FC
    table = jnp.concatenate(
        [uv, attr, jnp.zeros((V, 123), jnp.float32)], axis=1
    )
    idx = face_indices.astype(jnp.int32).T.reshape(-1)
    g = _sc_gather(table, idx).reshape(3, F, 128)
    vdata = jnp.transpose(g[:, :, :8], (0, 2, 1))
    vdata = jnp.transpose(vdata.reshape(3, 8, nc, FC), (0, 2, 1, 3))
    a0 = g[0, :, 2:5].reshape(nc, FC, 3)
    a1 = g[1, :, 2:5].reshape(nc, FC, 3)
    a2 = g[2, :, 2:5].reshape(nc, FC, 3)
    out = _bake(vdata, a0, a1, a2)
    return out.reshape(RES, RES, 3)


# speedup vs baseline: 7.2922x; 1.5589x over previous
import functools

import jax
import jax.numpy as jnp
from jax import lax
from jax.experimental import pallas as pl
from jax.experimental.pallas import tpu as pltpu
from jax.experimental.pallas import tpu_sc as plsc

RES = 256
P = RES * RES
TW = 16
TH = 16
PB = TW * TH
NTX = RES // TW
FC = 256
BIG = 1 << 30


def _sc_gather(table, idx):
    B = idx.shape[0]
    D = table.shape[1]
    n_workers = 32
    bpw = B // n_workers
    n_chunks = 2
    cw = bpw // n_chunks
    mesh = plsc.VectorSubcoreMesh(core_axis_name="c", subcore_axis_name="s")

    @functools.partial(
        pl.kernel,
        mesh=mesh,
        out_type=jax.ShapeDtypeStruct((B, D), jnp.float32),
        scratch_types=[
            pltpu.VMEM((n_chunks, cw), jnp.int32),
            pltpu.VMEM((cw, D), jnp.float32),
            pltpu.SemaphoreType.DMA,
        ],
    )
    def k(table_hbm, idx_hbm, out_hbm, idx_v, rows_v, sem):
        wid = lax.axis_index("s") * 2 + lax.axis_index("c")
        base = wid * bpw
        for j in range(n_chunks):
            pltpu.sync_copy(idx_hbm.at[pl.ds(base + j * cw, cw)], idx_v.at[j])
            pltpu.async_copy(table_hbm.at[idx_v.at[j]], rows_v, sem).wait()
            pltpu.sync_copy(rows_v, out_hbm.at[pl.ds(base + j * cw, cw)])

    return k(table, idx)


def _raster_body(
    vdata_ref, a0_ref, a1_ref, a2_ref, out_ref, fidx_s, acc_s, done_s
):
    nc = a0_ref.shape[0]
    fidx_s[...] = jnp.full((PB, 1), BIG, jnp.int32)
    acc_s[...] = jnp.zeros((PB, 3), jnp.float32)
    done_s[0] = 0

    pid = pl.program_id(0)
    ty = pid // NTX
    tx = pid % NTX
    li = lax.broadcasted_iota(jnp.int32, (PB, 1), 0)
    gx = tx * TW + li % TW
    gy = ty * TH + li // TW
    pxs = (gx.astype(jnp.float32) + 0.5) / float(RES)
    pys = (gy.astype(jnp.float32) + 0.5) / float(RES)

    def chunk(cidx, carry):
        @pl.when(done_s[0] == 0)
        def _():
            v0x = vdata_ref[0, cidx, 0:1, :]
            v0y = vdata_ref[0, cidx, 1:2, :]
            v1x = vdata_ref[1, cidx, 0:1, :]
            v1y = vdata_ref[1, cidx, 1:2, :]
            v2x = vdata_ref[2, cidx, 0:1, :]
            v2y = vdata_ref[2, cidx, 1:2, :]

            e0 = v1y - v2y
            e1 = v2x - v1x
            e2 = v2y - v0y
            e3 = v0x - v2x
            d = e0 * (v0x - v2x) + e1 * (v0y - v2y)
            absd = jnp.abs(d)
            valid = absd > 1e-8
            s = jnp.where(d >= 0.0, 1.0, -1.0)
            ku0 = s * e0
            ku1 = s * e1
            kv0 = s * e2
            kv1 = s * e3
            absd_x = jnp.where(valid, absd, -1.0)
            rd = jnp.where(valid, 1.0 / absd, 1.0)

            t0 = pxs - v2x
            t1 = pys - v2y
            su = ku0 * t0 + ku1 * t1
            sv = kv0 * t0 + kv1 * t1
            sw = absd_x - su - sv
            min3 = jnp.minimum(su, jnp.minimum(sv, sw))

            cols = lax.broadcasted_iota(jnp.int32, (PB, FC), 1)
            localmin = jnp.min(
                jnp.where(min3 >= 0.0, cols, BIG), axis=1, keepdims=True
            )
            fidxc = localmin + cidx * FC
            fold = fidx_s[...]
            upd = fidxc < fold
            ohf = jnp.where((cols == localmin) & upd, rd, 0.0)
            m0 = ohf * su
            m1 = ohf * sv
            m2 = ohf * sw

            acc = lax.dot(m0, a0_ref[cidx])
            acc += lax.dot(m1, a1_ref[cidx])
            acc += lax.dot(m2, a2_ref[cidx])
            acc_s[...] += acc
            fnew = jnp.where(upd, fidxc, fold)
            fidx_s[...] = fnew
            done_s[0] = (jnp.max(fnew) < BIG).astype(jnp.int32)

        return carry

    lax.fori_loop(0, nc, chunk, 0)
    out_ref[...] = acc_s[...]


def _bake(vdata, a0, a1, a2, interpret=False):
    nc = a0.shape[0]
    return pl.pallas_call(
        _raster_body,
        grid=(P // PB,),
        in_specs=[
            pl.BlockSpec((3, nc, 8, FC), lambda b: (0, 0, 0, 0)),
            pl.BlockSpec((nc, FC, 3), lambda b: (0, 0, 0)),
            pl.BlockSpec((nc, FC, 3), lambda b: (0, 0, 0)),
            pl.BlockSpec((nc, FC, 3), lambda b: (0, 0, 0)),
        ],
        out_specs=pl.BlockSpec((PB, 3), lambda b: (b, 0)),
        out_shape=jax.ShapeDtypeStruct((P, 3), jnp.float32),
        scratch_shapes=[
            pltpu.VMEM((PB, 1), jnp.int32),
            pltpu.VMEM((PB, 3), jnp.float32),
            pltpu.SMEM((1,), jnp.int32),
        ],
        compiler_params=pltpu.CompilerParams(
            dimension_semantics=("arbitrary",),
        ),
        interpret=interpret,
    )(vdata, a0, a1, a2)


def kernel(attr, uv, face_indices, bake_resolution, device):
    V = uv.shape[0]
    F = face_indices.shape[0]
    nc = F // FC
    table = jnp.concatenate(
        [uv, attr, jnp.zeros((V, 123), jnp.float32)], axis=1
    )
    idx = face_indices.astype(jnp.int32).T.reshape(-1)
    g = _sc_gather(table, idx).reshape(3, F, 128)
    vdata = jnp.transpose(g[:, :, :8], (0, 2, 1))
    vdata = jnp.transpose(vdata.reshape(3, 8, nc, FC), (0, 2, 1, 3))
    a0 = g[0, :, 2:5].reshape(nc, FC, 3)
    a1 = g[1, :, 2:5].reshape(nc, FC, 3)
    a2 = g[2, :, 2:5].reshape(nc, FC, 3)
    out = _bake(vdata, a0, a1, a2)
    out = out.reshape(RES // TH, NTX, TH, TW, 3).transpose(0, 2, 1, 3, 4)
    return out.reshape(RES, RES, 3)


# speedup vs baseline: 7.3665x; 1.0102x over previous
import functools

import jax
import jax.numpy as jnp
from jax import lax
from jax.experimental import pallas as pl
from jax.experimental.pallas import tpu as pltpu
from jax.experimental.pallas import tpu_sc as plsc

RES = 256
P = RES * RES
TW = 16
TH = 16
PB = TW * TH
NTX = RES // TW
CHUNKS = (128, 128, 256, 512, 1024)
BIG = 1 << 30


def _sc_gather(table, idx):
    B = idx.shape[0]
    D = table.shape[1]
    n_workers = 32
    bpw = B // n_workers
    n_chunks = 2
    cw = bpw // n_chunks
    mesh = plsc.VectorSubcoreMesh(core_axis_name="c", subcore_axis_name="s")

    @functools.partial(
        pl.kernel,
        mesh=mesh,
        out_type=jax.ShapeDtypeStruct((B, D), jnp.float32),
        scratch_types=[
            pltpu.VMEM((n_chunks, cw), jnp.int32),
            pltpu.VMEM((cw, D), jnp.float32),
            pltpu.SemaphoreType.DMA,
        ],
    )
    def k(table_hbm, idx_hbm, out_hbm, idx_v, rows_v, sem):
        wid = lax.axis_index("s") * 2 + lax.axis_index("c")
        base = wid * bpw
        for j in range(n_chunks):
            pltpu.sync_copy(idx_hbm.at[pl.ds(base + j * cw, cw)], idx_v.at[j])
            pltpu.async_copy(table_hbm.at[idx_v.at[j]], rows_v, sem).wait()
            pltpu.sync_copy(rows_v, out_hbm.at[pl.ds(base + j * cw, cw)])

    return k(table, idx)


def _raster_body(
    vdata_ref, a0_ref, a1_ref, a2_ref, out_ref, fidx_s, acc_s, done_s
):
    F = a0_ref.shape[0]
    fidx_s[...] = jnp.full((PB, 1), BIG, jnp.int32)
    acc_s[...] = jnp.zeros((PB, 3), jnp.float32)
    done_s[0] = 0

    pid = pl.program_id(0)
    ty = pid // NTX
    tx = pid % NTX
    li = lax.broadcasted_iota(jnp.int32, (PB, 1), 0)
    gx = tx * TW + li % TW
    gy = ty * TH + li // TW
    pxs = (gx.astype(jnp.float32) + 0.5) / float(RES)
    pys = (gy.astype(jnp.float32) + 0.5) / float(RES)

    o = 0
    for w in CHUNKS:
        off = o

        @pl.when(done_s[0] == 0)
        def _section(off=off, w=w):
            v0x = vdata_ref[0, 0:1, off:off + w]
            v0y = vdata_ref[0, 1:2, off:off + w]
            v1x = vdata_ref[1, 0:1, off:off + w]
            v1y = vdata_ref[1, 1:2, off:off + w]
            v2x = vdata_ref[2, 0:1, off:off + w]
            v2y = vdata_ref[2, 1:2, off:off + w]

            e0 = v1y - v2y
            e1 = v2x - v1x
            e2 = v2y - v0y
            e3 = v0x - v2x
            d = e0 * (v0x - v2x) + e1 * (v0y - v2y)
            absd = jnp.abs(d)
            valid = absd > 1e-8
            s = jnp.where(d >= 0.0, 1.0, -1.0)
            ku0 = s * e0
            ku1 = s * e1
            kv0 = s * e2
            kv1 = s * e3
            absd_x = jnp.where(valid, absd, -1.0)

            t0 = pxs - v2x
            t1 = pys - v2y
            su = ku0 * t0 + ku1 * t1
            sv = kv0 * t0 + kv1 * t1
            sw = absd_x - su - sv
            min3 = jnp.minimum(su, jnp.minimum(sv, sw))

            cols = lax.broadcasted_iota(jnp.int32, (PB, w), 1)
            localmin = jnp.min(
                jnp.where(min3 >= 0.0, cols, BIG), axis=1, keepdims=True
            )
            fold = fidx_s[...]
            upd = (localmin + off) < fold
            fidx_s[...] = jnp.where(upd, localmin + off, fold)
            done_s[0] = (jnp.max(jnp.where(upd, 0, fold)) < BIG).astype(
                jnp.int32
            )

            @pl.when(jnp.max(upd.astype(jnp.int32)) > 0)
            def _interp():
                rd = jnp.where(valid, 1.0 / absd, 1.0)
                ohf = jnp.where((cols == localmin) & upd, rd, 0.0)
                m0 = ohf * su
                m1 = ohf * sv
                m2 = ohf * sw
                acc = lax.dot(m0, a0_ref[off:off + w, :])
                acc += lax.dot(m1, a1_ref[off:off + w, :])
                acc += lax.dot(m2, a2_ref[off:off + w, :])
                acc_s[...] += acc

        o += w

    out_ref[...] = acc_s[...]


def _bake(vdata, a0, a1, a2, interpret=False):
    F = a0.shape[0]
    return pl.pallas_call(
        _raster_body,
        grid=(P // PB,),
        in_specs=[
            pl.BlockSpec((3, 8, F), lambda b: (0, 0, 0)),
            pl.BlockSpec((F, 3), lambda b: (0, 0)),
            pl.BlockSpec((F, 3), lambda b: (0, 0)),
            pl.BlockSpec((F, 3), lambda b: (0, 0)),
        ],
        out_specs=pl.BlockSpec((PB, 3), lambda b: (b, 0)),
        out_shape=jax.ShapeDtypeStruct((P, 3), jnp.float32),
        scratch_shapes=[
            pltpu.VMEM((PB, 1), jnp.int32),
            pltpu.VMEM((PB, 3), jnp.float32),
            pltpu.SMEM((1,), jnp.int32),
        ],
        compiler_params=pltpu.CompilerParams(
            dimension_semantics=("arbitrary",),
        ),
        interpret=interpret,
    )(vdata, a0, a1, a2)


def kernel(attr, uv, face_indices, bake_resolution, device):
    V = uv.shape[0]
    F = face_indices.shape[0]
    table = jnp.concatenate(
        [uv, attr, jnp.zeros((V, 123), jnp.float32)], axis=1
    )
    idx = face_indices.astype(jnp.int32).T.reshape(-1)
    g = _sc_gather(table, idx).reshape(3, F, 128)
    vdata = jnp.transpose(g[:, :, :8], (0, 2, 1))
    a0 = g[0, :, 2:5]
    a1 = g[1, :, 2:5]
    a2 = g[2, :, 2:5]
    out = _bake(vdata, a0, a1, a2)
    out = out.reshape(RES // TH, NTX, TH, TW, 3).transpose(0, 2, 1, 3, 4)
    return out.reshape(RES, RES, 3)


# speedup vs baseline: 7.3793x; 1.0017x over previous
import functools

import jax
import jax.numpy as jnp
from jax import lax
from jax.experimental import pallas as pl
from jax.experimental.pallas import tpu as pltpu
from jax.experimental.pallas import tpu_sc as plsc

RES = 256
P = RES * RES
TW = 16
TH = 16
PB = TW * TH
NTX = RES // TW
CHUNKS = (128, 128, 256, 512, 1024)
BIG = 1 << 30


def _sc_gather(table, idx):
    B = idx.shape[0]
    D = table.shape[1]
    n_workers = 32
    bpw = B // n_workers
    n_chunks = 2
    cw = bpw // n_chunks
    mesh = plsc.VectorSubcoreMesh(core_axis_name="c", subcore_axis_name="s")

    @functools.partial(
        pl.kernel,
        mesh=mesh,
        out_type=jax.ShapeDtypeStruct((B, D), jnp.float32),
        scratch_types=[
            pltpu.VMEM((n_chunks, cw), jnp.int32),
            pltpu.VMEM((cw, D), jnp.float32),
            pltpu.SemaphoreType.DMA,
        ],
    )
    def k(table_hbm, idx_hbm, out_hbm, idx_v, rows_v, sem):
        wid = lax.axis_index("s") * 2 + lax.axis_index("c")
        base = wid * bpw
        for j in range(n_chunks):
            pltpu.sync_copy(idx_hbm.at[pl.ds(base + j * cw, cw)], idx_v.at[j])
            pltpu.async_copy(table_hbm.at[idx_v.at[j]], rows_v, sem).wait()
            pltpu.sync_copy(rows_v, out_hbm.at[pl.ds(base + j * cw, cw)])

    return k(table, idx)


def _raster_body(
    vdata_ref, a0_ref, a1_ref, a2_ref, out_ref, fidx_s, acc_s, done_s, coef_s
):
    F = a0_ref.shape[0]
    pid = pl.program_id(0)

    @pl.when(pid == 0)
    def _coefs():
        v0x = vdata_ref[0, 0:1, :]
        v0y = vdata_ref[0, 1:2, :]
        v1x = vdata_ref[1, 0:1, :]
        v1y = vdata_ref[1, 1:2, :]
        v2x = vdata_ref[2, 0:1, :]
        v2y = vdata_ref[2, 1:2, :]
        e0 = v1y - v2y
        e1 = v2x - v1x
        e2 = v2y - v0y
        e3 = v0x - v2x
        d = e0 * (v0x - v2x) + e1 * (v0y - v2y)
        absd = jnp.abs(d)
        valid = absd > 1e-8
        s = jnp.where(d >= 0.0, 1.0, -1.0)
        coef_s[0:1, :] = s * e0
        coef_s[1:2, :] = s * e1
        coef_s[2:3, :] = s * e2
        coef_s[3:4, :] = s * e3
        coef_s[4:5, :] = v2x
        coef_s[5:6, :] = v2y
        coef_s[6:7, :] = jnp.where(valid, absd, -1.0)
        coef_s[7:8, :] = jnp.where(valid, 1.0 / absd, 1.0)

    fidx_s[...] = jnp.full((PB, 1), BIG, jnp.int32)
    acc_s[...] = jnp.zeros((PB, 3), jnp.float32)
    done_s[0] = 0

    ty = pid // NTX
    tx = pid % NTX
    li = lax.broadcasted_iota(jnp.int32, (PB, 1), 0)
    gx = tx * TW + li % TW
    gy = ty * TH + li // TW
    pxs = (gx.astype(jnp.float32) + 0.5) / float(RES)
    pys = (gy.astype(jnp.float32) + 0.5) / float(RES)

    o = 0
    for w in CHUNKS:
        off = o

        @pl.when(done_s[0] == 0)
        def _section(off=off, w=w):
            ku0 = coef_s[0:1, off:off + w]
            ku1 = coef_s[1:2, off:off + w]
            kv0 = coef_s[2:3, off:off + w]
            kv1 = coef_s[3:4, off:off + w]
            v2x = coef_s[4:5, off:off + w]
            v2y = coef_s[5:6, off:off + w]
            absd_x = coef_s[6:7, off:off + w]

            t0 = pxs - v2x
            t1 = pys - v2y
            su = ku0 * t0 + ku1 * t1
            sv = kv0 * t0 + kv1 * t1
            sw = absd_x - su - sv
            min3 = jnp.minimum(su, jnp.minimum(sv, sw))

            cols = lax.broadcasted_iota(jnp.int32, (PB, w), 1)
            localmin = jnp.min(
                jnp.where(min3 >= 0.0, cols, BIG), axis=1, keepdims=True
            )
            fold = fidx_s[...]
            upd = (localmin + off) < fold
            fidx_s[...] = jnp.where(upd, localmin + off, fold)
            done_s[0] = (jnp.max(jnp.where(upd, 0, fold)) < BIG).astype(
                jnp.int32
            )

            @pl.when(jnp.max(upd.astype(jnp.int32)) > 0)
            def _interp():
                rd = coef_s[7:8, off:off + w]
                ohf = jnp.where((cols == localmin) & upd, rd, 0.0)
                m0 = ohf * su
                m1 = ohf * sv
                m2 = ohf * sw
                acc = lax.dot(m0, a0_ref[off:off + w, :])
                acc += lax.dot(m1, a1_ref[off:off + w, :])
                acc += lax.dot(m2, a2_ref[off:off + w, :])
                acc_s[...] += acc

        o += w

    out_ref[...] = acc_s[...]


def _bake(vdata, a0, a1, a2, interpret=False):
    F = a0.shape[0]
    return pl.pallas_call(
        _raster_body,
        grid=(P // PB,),
        in_specs=[
            pl.BlockSpec((3, 8, F), lambda b: (0, 0, 0)),
            pl.BlockSpec((F, 3), lambda b: (0, 0)),
            pl.BlockSpec((F, 3), lambda b: (0, 0)),
            pl.BlockSpec((F, 3), lambda b: (0, 0)),
        ],
        out_specs=pl.BlockSpec((PB, 3), lambda b: (b, 0)),
        out_shape=jax.ShapeDtypeStruct((P, 3), jnp.float32),
        scratch_shapes=[
            pltpu.VMEM((PB, 1), jnp.int32),
            pltpu.VMEM((PB, 3), jnp.float32),
            pltpu.SMEM((1,), jnp.int32),
            pltpu.VMEM((8, F), jnp.float32),
        ],
        compiler_params=pltpu.CompilerParams(
            dimension_semantics=("arbitrary",),
        ),
        interpret=interpret,
    )(vdata, a0, a1, a2)


def kernel(attr, uv, face_indices, bake_resolution, device):
    V = uv.shape[0]
    F = face_indices.shape[0]
    table = jnp.concatenate(
        [uv, attr, jnp.zeros((V, 123), jnp.float32)], axis=1
    )
    idx = face_indices.astype(jnp.int32).T.reshape(-1)
    g = _sc_gather(table, idx).reshape(3, F, 128)
    vdata = jnp.transpose(g[:, :, :8], (0, 2, 1))
    a0 = g[0, :, 2:5]
    a1 = g[1, :, 2:5]
    a2 = g[2, :, 2:5]
    out = _bake(vdata, a0, a1, a2)
    out = out.reshape(RES // TH, NTX, TH, TW, 3).transpose(0, 2, 1, 3, 4)
    return out.reshape(RES, RES, 3)


# speedup vs baseline: 7.4344x; 1.0075x over previous
import functools

import jax
import jax.numpy as jnp
from jax import lax
from jax.experimental import pallas as pl
from jax.experimental.pallas import tpu as pltpu
from jax.experimental.pallas import tpu_sc as plsc

RES = 256
P = RES * RES
TW = 16
TH = 16
PB = TW * TH
NTX = RES // TW
CHUNKS = (128, 128, 256, 512, 1024)
TPG = 4
BIG = 1 << 30


def _sc_gather(table, idx):
    B = idx.shape[0]
    D = table.shape[1]
    n_workers = 32
    bpw = B // n_workers
    n_chunks = 2
    cw = bpw // n_chunks
    mesh = plsc.VectorSubcoreMesh(core_axis_name="c", subcore_axis_name="s")

    @functools.partial(
        pl.kernel,
        mesh=mesh,
        out_type=jax.ShapeDtypeStruct((B, D), jnp.float32),
        scratch_types=[
            pltpu.VMEM((n_chunks, cw), jnp.int32),
            pltpu.VMEM((cw, D), jnp.float32),
            pltpu.SemaphoreType.DMA,
        ],
    )
    def k(table_hbm, idx_hbm, out_hbm, idx_v, rows_v, sem):
        wid = lax.axis_index("s") * 2 + lax.axis_index("c")
        base = wid * bpw
        for j in range(n_chunks):
            pltpu.sync_copy(idx_hbm.at[pl.ds(base + j * cw, cw)], idx_v.at[j])
            pltpu.async_copy(table_hbm.at[idx_v.at[j]], rows_v, sem).wait()
            pltpu.sync_copy(rows_v, out_hbm.at[pl.ds(base + j * cw, cw)])

    return k(table, idx)


def _raster_body(
    vdata_ref, a0_ref, a1_ref, a2_ref, out_ref, fidx_s, acc_s, done_s, coef_s
):
    F = a0_ref.shape[0]
    pid = pl.program_id(0)

    @pl.when(pid == 0)
    def _coefs():
        v0x = vdata_ref[0, 0:1, :]
        v0y = vdata_ref[0, 1:2, :]
        v1x = vdata_ref[1, 0:1, :]
        v1y = vdata_ref[1, 1:2, :]
        v2x = vdata_ref[2, 0:1, :]
        v2y = vdata_ref[2, 1:2, :]
        e0 = v1y - v2y
        e1 = v2x - v1x
        e2 = v2y - v0y
        e3 = v0x - v2x
        d = e0 * (v0x - v2x) + e1 * (v0y - v2y)
        absd = jnp.abs(d)
        valid = absd > 1e-8
        s = jnp.where(d >= 0.0, 1.0, -1.0)
        coef_s[0:1, :] = s * e0
        coef_s[1:2, :] = s * e1
        coef_s[2:3, :] = s * e2
        coef_s[3:4, :] = s * e3
        coef_s[4:5, :] = v2x
        coef_s[5:6, :] = v2y
        coef_s[6:7, :] = jnp.where(valid, absd, -1.0)
        coef_s[7:8, :] = jnp.where(valid, 1.0 / absd, 1.0)

    for k in range(TPG):
        tid = pid * TPG + k
        fidx_s[...] = jnp.full((PB, 1), BIG, jnp.int32)
        acc_s[...] = jnp.zeros((PB, 3), jnp.float32)
        done_s[0] = 0

        ty = tid // NTX
        tx = tid % NTX
        li = lax.broadcasted_iota(jnp.int32, (PB, 1), 0)
        gx = tx * TW + li % TW
        gy = ty * TH + li // TW
        pxs = (gx.astype(jnp.float32) + 0.5) / float(RES)
        pys = (gy.astype(jnp.float32) + 0.5) / float(RES)

        o = 0
        for w in CHUNKS:
            off = o

            @pl.when(done_s[0] == 0)
            def _section(off=off, w=w, pxs=pxs, pys=pys):
                ku0 = coef_s[0:1, off:off + w]
                ku1 = coef_s[1:2, off:off + w]
                kv0 = coef_s[2:3, off:off + w]
                kv1 = coef_s[3:4, off:off + w]
                v2x = coef_s[4:5, off:off + w]
                v2y = coef_s[5:6, off:off + w]
                absd_x = coef_s[6:7, off:off + w]

                t0 = pxs - v2x
                t1 = pys - v2y
                su = ku0 * t0 + ku1 * t1
                sv = kv0 * t0 + kv1 * t1
                sw = absd_x - su - sv
                min3 = jnp.minimum(su, jnp.minimum(sv, sw))

                cols = lax.broadcasted_iota(jnp.int32, (PB, w), 1)
                localmin = jnp.min(
                    jnp.where(min3 >= 0.0, cols, BIG), axis=1, keepdims=True
                )
                fold = fidx_s[...]
                upd = (localmin + off) < fold
                fidx_s[...] = jnp.where(upd, localmin + off, fold)
                done_s[0] = (jnp.max(jnp.where(upd, 0, fold)) < BIG).astype(
                    jnp.int32
                )

                @pl.when(jnp.max(upd.astype(jnp.int32)) > 0)
                def _interp():
                    rd = coef_s[7:8, off:off + w]
                    ohf = jnp.where((cols == localmin) & upd, rd, 0.0)
                    m0 = ohf * su
                    m1 = ohf * sv
                    m2 = ohf * sw
                    acc = lax.dot(m0, a0_ref[off:off + w, :])
                    acc += lax.dot(m1, a1_ref[off:off + w, :])
                    acc += lax.dot(m2, a2_ref[off:off + w, :])
                    acc_s[...] += acc

            o += w

        out_ref[k * PB:(k + 1) * PB, :] = acc_s[...]


def _bake(vdata, a0, a1, a2, interpret=False):
    F = a0.shape[0]
    return pl.pallas_call(
        _raster_body,
        grid=(P // (PB * TPG),),
        in_specs=[
            pl.BlockSpec((3, 8, F), lambda b: (0, 0, 0)),
            pl.BlockSpec((F, 3), lambda b: (0, 0)),
            pl.BlockSpec((F, 3), lambda b: (0, 0)),
            pl.BlockSpec((F, 3), lambda b: (0, 0)),
        ],
        out_specs=pl.BlockSpec((PB * TPG, 3), lambda b: (b, 0)),
        out_shape=jax.ShapeDtypeStruct((P, 3), jnp.float32),
        scratch_shapes=[
            pltpu.VMEM((PB, 1), jnp.int32),
            pltpu.VMEM((PB, 3), jnp.float32),
            pltpu.SMEM((1,), jnp.int32),
            pltpu.VMEM((8, F), jnp.float32),
        ],
        compiler_params=pltpu.CompilerParams(
            dimension_semantics=("arbitrary",),
        ),
        interpret=interpret,
    )(vdata, a0, a1, a2)


def kernel(attr, uv, face_indices, bake_resolution, device):
    V = uv.shape[0]
    F = face_indices.shape[0]
    table = jnp.concatenate(
        [uv, attr, jnp.zeros((V, 123), jnp.float32)], axis=1
    )
    idx = face_indices.astype(jnp.int32).T.reshape(-1)
    g = _sc_gather(table, idx).reshape(3, F, 128)
    vdata = jnp.transpose(g[:, :, :8], (0, 2, 1))
    a0 = g[0, :, 2:5]
    a1 = g[1, :, 2:5]
    a2 = g[2, :, 2:5]
    out = _bake(vdata, a0, a1, a2)
    out = out.reshape(RES // TH, NTX, TH, TW, 3).transpose(0, 2, 1, 3, 4)
    return out.reshape(RES, RES, 3)


# speedup vs baseline: 8.1411x; 1.0951x over previous
import functools

import jax
import jax.numpy as jnp
from jax import lax
from jax.experimental import pallas as pl
from jax.experimental.pallas import tpu as pltpu
from jax.experimental.pallas import tpu_sc as plsc

RES = 256
P = RES * RES
TW = 16
TH = 16
PB = TW * TH
NTX = RES // TW
CHUNKS = (128, 128, 1792)
TPG = 4
BIG = 1 << 30


def _sc_gather(table, idx):
    B = idx.shape[0]
    D = table.shape[1]
    n_workers = 32
    bpw = B // n_workers
    n_chunks = 2
    cw = bpw // n_chunks
    mesh = plsc.VectorSubcoreMesh(core_axis_name="c", subcore_axis_name="s")

    @functools.partial(
        pl.kernel,
        mesh=mesh,
        out_type=jax.ShapeDtypeStruct((B, D), jnp.float32),
        scratch_types=[
            pltpu.VMEM((n_chunks, cw), jnp.int32),
            pltpu.VMEM((cw, D), jnp.float32),
            pltpu.SemaphoreType.DMA,
        ],
    )
    def k(table_hbm, idx_hbm, out_hbm, idx_v, rows_v, sem):
        wid = lax.axis_index("s") * 2 + lax.axis_index("c")
        base = wid * bpw
        for j in range(n_chunks):
            pltpu.sync_copy(idx_hbm.at[pl.ds(base + j * cw, cw)], idx_v.at[j])
            pltpu.async_copy(table_hbm.at[idx_v.at[j]], rows_v, sem).wait()
            pltpu.sync_copy(rows_v, out_hbm.at[pl.ds(base + j * cw, cw)])

    return k(table, idx)


def _raster_body(
    vdata_ref, a0_ref, a1_ref, a2_ref, out_ref, fidx_s, acc_s, done_s, coef_s
):
    F = a0_ref.shape[0]
    pid = pl.program_id(0)

    @pl.when(pid == 0)
    def _coefs():
        v0x = vdata_ref[0, 0:1, :]
        v0y = vdata_ref[0, 1:2, :]
        v1x = vdata_ref[1, 0:1, :]
        v1y = vdata_ref[1, 1:2, :]
        v2x = vdata_ref[2, 0:1, :]
        v2y = vdata_ref[2, 1:2, :]
        e0 = v1y - v2y
        e1 = v2x - v1x
        e2 = v2y - v0y
        e3 = v0x - v2x
        d = e0 * (v0x - v2x) + e1 * (v0y - v2y)
        absd = jnp.abs(d)
        valid = absd > 1e-8
        s = jnp.where(d >= 0.0, 1.0, -1.0)
        coef_s[0:1, :] = s * e0
        coef_s[1:2, :] = s * e1
        coef_s[2:3, :] = s * e2
        coef_s[3:4, :] = s * e3
        coef_s[4:5, :] = v2x
        coef_s[5:6, :] = v2y
        coef_s[6:7, :] = jnp.where(valid, absd, -1.0)
        coef_s[7:8, :] = jnp.where(valid, 1.0 / absd, 1.0)

    for k in range(TPG):
        tid = pid * TPG + k
        fidx_s[...] = jnp.full((PB, 1), BIG, jnp.int32)
        acc_s[...] = jnp.zeros((PB, 3), jnp.float32)
        done_s[0] = 0

        ty = tid // NTX
        tx = tid % NTX
        li = lax.broadcasted_iota(jnp.int32, (PB, 1), 0)
        gx = tx * TW + li % TW
        gy = ty * TH + li // TW
        pxs = (gx.astype(jnp.float32) + 0.5) / float(RES)
        pys = (gy.astype(jnp.float32) + 0.5) / float(RES)

        o = 0
        for w in CHUNKS:
            off = o

            @pl.when(done_s[0] == 0)
            def _section(off=off, w=w, pxs=pxs, pys=pys):
                ku0 = coef_s[0:1, off:off + w]
                ku1 = coef_s[1:2, off:off + w]
                kv0 = coef_s[2:3, off:off + w]
                kv1 = coef_s[3:4, off:off + w]
                v2x = coef_s[4:5, off:off + w]
                v2y = coef_s[5:6, off:off + w]
                absd_x = coef_s[6:7, off:off + w]

                t0 = pxs - v2x
                t1 = pys - v2y
                su = ku0 * t0 + ku1 * t1
                sv = kv0 * t0 + kv1 * t1
                sw = absd_x - su - sv
                min3 = jnp.minimum(su, jnp.minimum(sv, sw))

                cols = lax.broadcasted_iota(jnp.int32, (PB, w), 1)
                localmin = jnp.min(
                    jnp.where(min3 >= 0.0, cols, BIG), axis=1, keepdims=True
                )
                fold = fidx_s[...]
                upd = (localmin + off) < fold
                fidx_s[...] = jnp.where(upd, localmin + off, fold)
                done_s[0] = (jnp.max(jnp.where(upd, 0, fold)) < BIG).astype(
                    jnp.int32
                )

                @pl.when(jnp.max(upd.astype(jnp.int32)) > 0)
                def _interp():
                    rd = coef_s[7:8, off:off + w]
                    ohf = jnp.where((cols == localmin) & upd, rd, 0.0)
                    m0 = ohf * su
                    m1 = ohf * sv
                    m2 = ohf * sw
                    acc = lax.dot(m0, a0_ref[off:off + w, :])
                    acc += lax.dot(m1, a1_ref[off:off + w, :])
                    acc += lax.dot(m2, a2_ref[off:off + w, :])
                    acc_s[...] += acc

            o += w

        out_ref[k * PB:(k + 1) * PB, :] = acc_s[...]


def _bake(vdata, a0, a1, a2, interpret=False):
    F = a0.shape[0]
    return pl.pallas_call(
        _raster_body,
        grid=(P // (PB * TPG),),
        in_specs=[
            pl.BlockSpec((3, 8, F), lambda b: (0, 0, 0)),
            pl.BlockSpec((F, 3), lambda b: (0, 0)),
            pl.BlockSpec((F, 3), lambda b: (0, 0)),
            pl.BlockSpec((F, 3), lambda b: (0, 0)),
        ],
        out_specs=pl.BlockSpec((PB * TPG, 3), lambda b: (b, 0)),
        out_shape=jax.ShapeDtypeStruct((P, 3), jnp.float32),
        scratch_shapes=[
            pltpu.VMEM((PB, 1), jnp.int32),
            pltpu.VMEM((PB, 3), jnp.float32),
            pltpu.SMEM((1,), jnp.int32),
            pltpu.VMEM((8, F), jnp.float32),
        ],
        compiler_params=pltpu.CompilerParams(
            dimension_semantics=("arbitrary",),
        ),
        interpret=interpret,
    )(vdata, a0, a1, a2)


def kernel(attr, uv, face_indices, bake_resolution, device):
    V = uv.shape[0]
    F = face_indices.shape[0]
    table = jnp.concatenate(
        [uv, attr, jnp.zeros((V, 123), jnp.float32)], axis=1
    )
    idx = face_indices.astype(jnp.int32).T.reshape(-1)
    g = _sc_gather(table, idx).reshape(3, F, 128)
    vdata = jnp.transpose(g[:, :, :8], (0, 2, 1))
    a0 = g[0, :, 2:5]
    a1 = g[1, :, 2:5]
    a2 = g[2, :, 2:5]
    out = _bake(vdata, a0, a1, a2)
    out = out.reshape(RES // TH, NTX, TH, TW, 3).transpose(0, 2, 1, 3, 4)
    return out.reshape(RES, RES, 3)


# speedup vs baseline: 8.3169x; 1.0216x over previous
import functools

import jax
import jax.numpy as jnp
from jax import lax
from jax.experimental import pallas as pl
from jax.experimental.pallas import tpu as pltpu
from jax.experimental.pallas import tpu_sc as plsc

RES = 256
P = RES * RES
TW = 16
TH = 16
PB = TW * TH
NTX = RES // TW
CHUNKS = (128, 1920)
TPG = 4
BIG = 1 << 30


def _sc_gather(table, idx):
    B = idx.shape[0]
    D = table.shape[1]
    n_workers = 32
    bpw = B // n_workers
    n_chunks = 2
    cw = bpw // n_chunks
    mesh = plsc.VectorSubcoreMesh(core_axis_name="c", subcore_axis_name="s")

    @functools.partial(
        pl.kernel,
        mesh=mesh,
        out_type=jax.ShapeDtypeStruct((B, D), jnp.float32),
        scratch_types=[
            pltpu.VMEM((n_chunks, cw), jnp.int32),
            pltpu.VMEM((cw, D), jnp.float32),
            pltpu.SemaphoreType.DMA,
        ],
    )
    def k(table_hbm, idx_hbm, out_hbm, idx_v, rows_v, sem):
        wid = lax.axis_index("s") * 2 + lax.axis_index("c")
        base = wid * bpw
        for j in range(n_chunks):
            pltpu.sync_copy(idx_hbm.at[pl.ds(base + j * cw, cw)], idx_v.at[j])
            pltpu.async_copy(table_hbm.at[idx_v.at[j]], rows_v, sem).wait()
            pltpu.sync_copy(rows_v, out_hbm.at[pl.ds(base + j * cw, cw)])

    return k(table, idx)


def _raster_body(
    vdata_ref, a0_ref, a1_ref, a2_ref, out_ref, fidx_s, acc_s, done_s, coef_s
):
    F = a0_ref.shape[0]
    pid = pl.program_id(0)

    @pl.when(pid == 0)
    def _coefs():
        v0x = vdata_ref[0, 0:1, :]
        v0y = vdata_ref[0, 1:2, :]
        v1x = vdata_ref[1, 0:1, :]
        v1y = vdata_ref[1, 1:2, :]
        v2x = vdata_ref[2, 0:1, :]
        v2y = vdata_ref[2, 1:2, :]
        e0 = v1y - v2y
        e1 = v2x - v1x
        e2 = v2y - v0y
        e3 = v0x - v2x
        d = e0 * (v0x - v2x) + e1 * (v0y - v2y)
        absd = jnp.abs(d)
        valid = absd > 1e-8
        s = jnp.where(d >= 0.0, 1.0, -1.0)
        coef_s[0:1, :] = s * e0
        coef_s[1:2, :] = s * e1
        coef_s[2:3, :] = s * e2
        coef_s[3:4, :] = s * e3
        coef_s[4:5, :] = v2x
        coef_s[5:6, :] = v2y
        coef_s[6:7, :] = jnp.where(valid, absd, -1.0)
        coef_s[7:8, :] = jnp.where(valid, 1.0 / absd, 1.0)

    for k in range(TPG):
        tid = pid * TPG + k
        fidx_s[...] = jnp.full((PB, 1), BIG, jnp.int32)
        acc_s[...] = jnp.zeros((PB, 3), jnp.float32)
        done_s[0] = 0

        ty = tid // NTX
        tx = tid % NTX
        li = lax.broadcasted_iota(jnp.int32, (PB, 1), 0)
        gx = tx * TW + li % TW
        gy = ty * TH + li // TW
        pxs = (gx.astype(jnp.float32) + 0.5) / float(RES)
        pys = (gy.astype(jnp.float32) + 0.5) / float(RES)

        o = 0
        for w in CHUNKS:
            off = o

            @pl.when(done_s[0] == 0)
            def _section(off=off, w=w, pxs=pxs, pys=pys):
                ku0 = coef_s[0:1, off:off + w]
                ku1 = coef_s[1:2, off:off + w]
                kv0 = coef_s[2:3, off:off + w]
                kv1 = coef_s[3:4, off:off + w]
                v2x = coef_s[4:5, off:off + w]
                v2y = coef_s[5:6, off:off + w]
                absd_x = coef_s[6:7, off:off + w]

                t0 = pxs - v2x
                t1 = pys - v2y
                su = ku0 * t0 + ku1 * t1
                sv = kv0 * t0 + kv1 * t1
                sw = absd_x - su - sv
                min3 = jnp.minimum(su, jnp.minimum(sv, sw))

                cols = lax.broadcasted_iota(jnp.int32, (PB, w), 1)
                localmin = jnp.min(
                    jnp.where(min3 >= 0.0, cols, BIG), axis=1, keepdims=True
                )
                fold = fidx_s[...]
                upd = (localmin + off) < fold
                fidx_s[...] = jnp.where(upd, localmin + off, fold)
                done_s[0] = (jnp.max(jnp.where(upd, 0, fold)) < BIG).astype(
                    jnp.int32
                )

                @pl.when(jnp.max(upd.astype(jnp.int32)) > 0)
                def _interp():
                    rd = coef_s[7:8, off:off + w]
                    ohf = jnp.where((cols == localmin) & upd, rd, 0.0)
                    m0 = ohf * su
                    m1 = ohf * sv
                    m2 = ohf * sw
                    acc = lax.dot(m0, a0_ref[off:off + w, :])
                    acc += lax.dot(m1, a1_ref[off:off + w, :])
                    acc += lax.dot(m2, a2_ref[off:off + w, :])
                    acc_s[...] += acc

            o += w

        out_ref[k * PB:(k + 1) * PB, :] = acc_s[...]


def _bake(vdata, a0, a1, a2, interpret=False):
    F = a0.shape[0]
    return pl.pallas_call(
        _raster_body,
        grid=(P // (PB * TPG),),
        in_specs=[
            pl.BlockSpec((3, 8, F), lambda b: (0, 0, 0)),
            pl.BlockSpec((F, 3), lambda b: (0, 0)),
            pl.BlockSpec((F, 3), lambda b: (0, 0)),
            pl.BlockSpec((F, 3), lambda b: (0, 0)),
        ],
        out_specs=pl.BlockSpec((PB * TPG, 3), lambda b: (b, 0)),
        out_shape=jax.ShapeDtypeStruct((P, 3), jnp.float32),
        scratch_shapes=[
            pltpu.VMEM((PB, 1), jnp.int32),
            pltpu.VMEM((PB, 3), jnp.float32),
            pltpu.SMEM((1,), jnp.int32),
            pltpu.VMEM((8, F), jnp.float32),
        ],
        compiler_params=pltpu.CompilerParams(
            dimension_semantics=("arbitrary",),
        ),
        interpret=interpret,
    )(vdata, a0, a1, a2)


def kernel(attr, uv, face_indices, bake_resolution, device):
    V = uv.shape[0]
    F = face_indices.shape[0]
    table = jnp.concatenate(
        [uv, attr, jnp.zeros((V, 123), jnp.float32)], axis=1
    )
    idx = face_indices.astype(jnp.int32).T.reshape(-1)
    g = _sc_gather(table, idx).reshape(3, F, 128)
    vdata = jnp.transpose(g[:, :, :8], (0, 2, 1))
    a0 = g[0, :, 2:5]
    a1 = g[1, :, 2:5]
    a2 = g[2, :, 2:5]
    out = _bake(vdata, a0, a1, a2)
    out = out.reshape(RES // TH, NTX, TH, TW, 3).transpose(0, 2, 1, 3, 4)
    return out.reshape(RES, RES, 3)


# speedup vs baseline: 10.2455x; 1.2319x over previous
import functools

import jax
import jax.numpy as jnp
from jax import lax
from jax.experimental import pallas as pl
from jax.experimental.pallas import tpu as pltpu
from jax.experimental.pallas import tpu_sc as plsc

RES = 256
P = RES * RES
TW = 16
TH = 16
PB = TW * TH
NTX = RES // TW
CHUNKS = (128, 1920)
TPG = 4
BIG = 1 << 30


def _sc_gather(table, idx):
    B = idx.shape[0]
    D = table.shape[1]
    n_workers = 32
    bpw = B // n_workers
    n_chunks = 2
    cw = bpw // n_chunks
    mesh = plsc.VectorSubcoreMesh(core_axis_name="c", subcore_axis_name="s")

    @functools.partial(
        pl.kernel,
        mesh=mesh,
        out_type=jax.ShapeDtypeStruct((B, D), jnp.float32),
        scratch_types=[
            pltpu.VMEM((n_chunks, cw), jnp.int32),
            pltpu.VMEM((cw, D), jnp.float32),
            pltpu.SemaphoreType.DMA,
        ],
    )
    def k(table_hbm, idx_hbm, out_hbm, idx_v, rows_v, sem):
        wid = lax.axis_index("s") * 2 + lax.axis_index("c")
        base = wid * bpw
        for j in range(n_chunks):
            pltpu.sync_copy(idx_hbm.at[pl.ds(base + j * cw, cw)], idx_v.at[j])
            pltpu.async_copy(table_hbm.at[idx_v.at[j]], rows_v, sem).wait()
            pltpu.sync_copy(rows_v, out_hbm.at[pl.ds(base + j * cw, cw)])

    return k(table, idx)


def _raster_body(
    vdata_ref, a0_ref, a1_ref, a2_ref, out_ref, fidx_s, acc_s, done_s, coef_s
):
    F = a0_ref.shape[0]
    pid = pl.program_id(0)

    @pl.when(pid == 0)
    def _coefs():
        v0x = vdata_ref[0, 0:1, :]
        v0y = vdata_ref[0, 1:2, :]
        v1x = vdata_ref[1, 0:1, :]
        v1y = vdata_ref[1, 1:2, :]
        v2x = vdata_ref[2, 0:1, :]
        v2y = vdata_ref[2, 1:2, :]
        e0 = v1y - v2y
        e1 = v2x - v1x
        e2 = v2y - v0y
        e3 = v0x - v2x
        d = e0 * (v0x - v2x) + e1 * (v0y - v2y)
        absd = jnp.abs(d)
        valid = absd > 1e-8
        s = jnp.where(d >= 0.0, 1.0, -1.0)
        coef_s[0:1, :] = s * e0
        coef_s[1:2, :] = s * e1
        coef_s[2:3, :] = s * e2
        coef_s[3:4, :] = s * e3
        coef_s[4:5, :] = v2x
        coef_s[5:6, :] = v2y
        coef_s[6:7, :] = jnp.where(valid, absd, -1.0)
        coef_s[7:8, :] = jnp.where(valid, 1.0 / absd, 1.0)

    for k in range(TPG):
        tid = pid * TPG + k
        ty = tid // NTX
        tx = tid % NTX
        li = lax.broadcasted_iota(jnp.int32, (PB, 1), 0)
        gx = tx * TW + li % TW
        gy = ty * TH + li // TW
        pxs = (gx.astype(jnp.float32) + 0.5) / float(RES)
        pys = (gy.astype(jnp.float32) + 0.5) / float(RES)

        o = 0
        for w in CHUNKS:
            off = o

            @pl.when((off == 0) | (done_s[0] == 0))
            def _section(off=off, w=w, pxs=pxs, pys=pys):
                ku0 = coef_s[0:1, off:off + w]
                ku1 = coef_s[1:2, off:off + w]
                kv0 = coef_s[2:3, off:off + w]
                kv1 = coef_s[3:4, off:off + w]
                v2x = coef_s[4:5, off:off + w]
                v2y = coef_s[5:6, off:off + w]
                absd_x = coef_s[6:7, off:off + w]

                t0 = pxs - v2x
                t1 = pys - v2y
                su = ku0 * t0 + ku1 * t1
                sv = kv0 * t0 + kv1 * t1
                sw = absd_x - su - sv
                min3 = jnp.minimum(su, jnp.minimum(sv, sw))

                cols = lax.broadcasted_iota(jnp.int32, (PB, w), 1)
                localmin = jnp.min(
                    jnp.where(min3 >= 0.0, cols, BIG), axis=1, keepdims=True
                )
                rd = coef_s[7:8, off:off + w]
                if off == 0:
                    fidx_s[...] = localmin
                    done_s[0] = (jnp.max(localmin) < BIG).astype(jnp.int32)
                    ohf = jnp.where(cols == localmin, rd, 0.0)
                    m0 = ohf * su
                    m1 = ohf * sv
                    m2 = ohf * sw
                    acc = lax.dot(m0, a0_ref[off:off + w, :])
                    acc += lax.dot(m1, a1_ref[off:off + w, :])
                    acc += lax.dot(m2, a2_ref[off:off + w, :])
                    acc_s[...] = acc
                else:
                    fold = fidx_s[...]
                    upd = (localmin + off) < fold
                    fidx_s[...] = jnp.where(upd, localmin + off, fold)
                    done_s[0] = (
                        jnp.max(jnp.where(upd, 0, fold)) < BIG
                    ).astype(jnp.int32)

                    @pl.when(jnp.max(upd.astype(jnp.int32)) > 0)
                    def _interp():
                        ohf = jnp.where((cols == localmin) & upd, rd, 0.0)
                        m0 = ohf * su
                        m1 = ohf * sv
                        m2 = ohf * sw
                        acc = lax.dot(m0, a0_ref[off:off + w, :])
                        acc += lax.dot(m1, a1_ref[off:off + w, :])
                        acc += lax.dot(m2, a2_ref[off:off + w, :])
                        acc_s[...] += acc

            o += w

        out_ref[k * PB:(k + 1) * PB, :] = acc_s[...]


def _bake(vdata, a0, a1, a2, interpret=False):
    F = a0.shape[0]
    return pl.pallas_call(
        _raster_body,
        grid=(P // (PB * TPG),),
        in_specs=[
            pl.BlockSpec((3, 8, F), lambda b: (0, 0, 0)),
            pl.BlockSpec((F, 3), lambda b: (0, 0)),
            pl.BlockSpec((F, 3), lambda b: (0, 0)),
            pl.BlockSpec((F, 3), lambda b: (0, 0)),
        ],
        out_specs=pl.BlockSpec((PB * TPG, 3), lambda b: (b, 0)),
        out_shape=jax.ShapeDtypeStruct((P, 3), jnp.float32),
        scratch_shapes=[
            pltpu.VMEM((PB, 1), jnp.int32),
            pltpu.VMEM((PB, 3), jnp.float32),
            pltpu.SMEM((1,), jnp.int32),
            pltpu.VMEM((8, F), jnp.float32),
        ],
        compiler_params=pltpu.CompilerParams(
            dimension_semantics=("arbitrary",),
        ),
        interpret=interpret,
    )(vdata, a0, a1, a2)


def kernel(attr, uv, face_indices, bake_resolution, device):
    V = uv.shape[0]
    F = face_indices.shape[0]
    table = jnp.concatenate(
        [uv, attr, jnp.zeros((V, 123), jnp.float32)], axis=1
    )
    idx = face_indices.astype(jnp.int32).T.reshape(-1)
    g = _sc_gather(table, idx).reshape(3, F, 128)
    vdata = jnp.transpose(g[:, :, :8], (0, 2, 1))
    a0 = g[0, :, 2:5]
    a1 = g[1, :, 2:5]
    a2 = g[2, :, 2:5]
    out = _bake(vdata, a0, a1, a2)
    out = out.reshape(RES // TH, NTX, TH, TW, 3).transpose(0, 2, 1, 3, 4)
    return out.reshape(RES, RES, 3)
